# baseline (speedup 1.0000x reference)
import functools

import jax
import jax.numpy as jnp
from jax import lax
from jax.experimental import pallas as pl
from jax.experimental.pallas import tpu as pltpu
from jax.experimental.pallas import tpu_sc as plsc

EMB_DIM = 32
NUM_DOMAINS = 10
NUM_NEG = 5
NUM_TILES = 32
NB = 128
LANES = 16


def _make_sc_kernel(n_rows):
    rows_per_tile = n_rows // NUM_TILES
    n_blocks = rows_per_tile // NB
    slots = NB * NUM_DOMAINS

    @functools.partial(
        pl.kernel,
        out_type=jax.ShapeDtypeStruct((n_rows,), jnp.float32),
        mesh=plsc.VectorSubcoreMesh(core_axis_name="c", subcore_axis_name="s"),
        compiler_params=pltpu.CompilerParams(use_tc_tiling_on_sc=False),
        scratch_types=[
            pltpu.VMEM((NB, 128), jnp.int32),
            pltpu.VMEM((NB, LANES, EMB_DIM), jnp.float32),
            pltpu.VMEM((2, LANES), jnp.float32),
            pltpu.VMEM((NB,), jnp.float32),
            pltpu.SemaphoreType.DMA,
        ],
    )
    def sc_kernel(idx_hbm, table_hbm, wc_hbm, out_hbm, idx_v, rows_v, wc_v,
                  sc_v, sem):
        wid = lax.axis_index("s") * 2 + lax.axis_index("c")
        pltpu.sync_copy(wc_hbm, wc_v)
        w_row = wc_v[0, :]
        c_row = wc_v[1, :]
        half_expw = jnp.exp(w_row) * 0.5

        def block_body(b, carry):
            blk = wid * n_blocks + b
            pltpu.sync_copy(idx_hbm.at[pl.ds(blk * NB, NB)], idx_v)
            copies = [
                pltpu.async_copy(
                    table_hbm.at[idx_v.at[r, pl.ds(0, LANES)]],
                    rows_v.at[r], sem)
                for r in range(NB)
            ]
            for cp in copies:
                cp.wait()

            lane = lax.iota(jnp.int32, LANES)

            def group_body(g, c2):
                def row_body(ii, acc):
                    row = g * LANES + ii
                    v0 = rows_v[row, 0, 0:LANES]
                    v1 = rows_v[row, 0, LANES:EMB_DIM]
                    s0 = v0
                    s1 = v1
                    q = v0 * v0 + v1 * v1
                    for j in range(1, NUM_DOMAINS):
                        v0 = rows_v[row, j, 0:LANES]
                        v1 = rows_v[row, j, LANES:EMB_DIM]
                        s0 = s0 + v0
                        s1 = s1 + v1
                        q = q + v0 * v0 + v1 * v1
                    t = s0 * s0 + s1 * s1 - q
                    for step in (8, 4, 2, 1):
                        t = t + t.at[lane ^ step].get(
                            mode="promise_in_bounds")
                    return jnp.where(lane == ii, t, acc)

                acc = lax.fori_loop(0, LANES, row_body,
                                    jnp.zeros((LANES,), jnp.float32))
                sc_v[pl.ds(g * LANES, LANES)] = jnp.exp(
                    acc * half_expw + c_row)
                return c2

            lax.fori_loop(0, NB // LANES, group_body, 0)
            pltpu.sync_copy(sc_v, out_hbm.at[pl.ds(blk * NB, NB)])
            return carry

        lax.fori_loop(0, n_blocks, block_body, 0)

    return sc_kernel


def kernel(pos_x, neg_x, emb_table, pair_w0, c):
    b = pos_x.shape[0]
    pos128 = jnp.pad(pos_x, ((0, 0), (0, 128 - NUM_DOMAINS)), mode="edge")
    neg128 = jnp.pad(neg_x, ((0, 0), (0, 0), (0, 128 - NUM_DOMAINS)),
                     mode="edge")
    x128 = jnp.concatenate(
        [pos128, neg128.reshape(b * NUM_NEG, 128)], axis=0)
    n_rows = x128.shape[0]
    wc = jnp.stack([
        jnp.broadcast_to(pair_w0[0], (LANES,)),
        jnp.broadcast_to(c[0], (LANES,)),
    ]).astype(jnp.float32)
    scores = _make_sc_kernel(n_rows)(x128, emb_table, wc)
    pos_score = scores[:b]
    neg_score = scores[b:].reshape(b, NUM_NEG)
    return pos_score, neg_score


# speedup vs baseline: 1.0333x; 1.0333x over previous
import functools

import jax
import jax.numpy as jnp
from jax import lax
from jax.experimental import pallas as pl
from jax.experimental.pallas import tpu as pltpu
from jax.experimental.pallas import tpu_sc as plsc

EMB_DIM = 32
NUM_DOMAINS = 10
NUM_NEG = 5
NUM_TILES = 32
NB = 128
LANES = 16


def _make_sc_kernel(n_rows):
    rows_per_tile = n_rows // NUM_TILES
    n_blocks = rows_per_tile // NB
    slots = NB * NUM_DOMAINS

    @functools.partial(
        pl.kernel,
        out_type=jax.ShapeDtypeStruct((n_rows,), jnp.float32),
        mesh=plsc.VectorSubcoreMesh(core_axis_name="c", subcore_axis_name="s"),
        compiler_params=pltpu.CompilerParams(use_tc_tiling_on_sc=False),
        scratch_types=[
            pltpu.VMEM((NB * 128,), jnp.int32),
            pltpu.VMEM((NB, NUM_DOMAINS, EMB_DIM), jnp.float32),
            pltpu.VMEM((2, LANES), jnp.float32),
            pltpu.VMEM((NB,), jnp.float32),
            pltpu.SemaphoreType.DMA,
        ],
    )
    def sc_kernel(idx_hbm, table_hbm, wc_hbm, out_hbm, idx_v, rows_v, wc_v,
                  sc_v, sem):
        wid = lax.axis_index("s") * 2 + lax.axis_index("c")
        pltpu.sync_copy(wc_hbm, wc_v)
        w_row = wc_v[0, :]
        c_row = wc_v[1, :]
        half_expw = jnp.exp(w_row) * 0.5

        def block_body(b, carry):
            blk = wid * n_blocks + b
            pltpu.sync_copy(idx_hbm.at[pl.ds(blk * NB * 128, NB * 128)], idx_v)
            copies = [
                pltpu.async_copy(
                    table_hbm.at[idx_v.at[pl.ds(r * 128, NUM_DOMAINS)]],
                    rows_v.at[r], sem)
                for r in range(NB)
            ]
            for cp in copies:
                cp.wait()

            lane = lax.iota(jnp.int32, LANES)

            def group_body(g, c2):
                def row_body(ii, acc):
                    row = g * LANES + ii
                    v0 = rows_v[row, 0, 0:LANES]
                    v1 = rows_v[row, 0, LANES:EMB_DIM]
                    s0 = v0
                    s1 = v1
                    q = v0 * v0 + v1 * v1
                    for j in range(1, NUM_DOMAINS):
                        v0 = rows_v[row, j, 0:LANES]
                        v1 = rows_v[row, j, LANES:EMB_DIM]
                        s0 = s0 + v0
                        s1 = s1 + v1
                        q = q + v0 * v0 + v1 * v1
                    t = s0 * s0 + s1 * s1 - q
                    for step in (8, 4, 2, 1):
                        t = t + t.at[lane ^ step].get(
                            mode="promise_in_bounds")
                    return jnp.where(lane == ii, t, acc)

                acc = lax.fori_loop(0, LANES, row_body,
                                    jnp.zeros((LANES,), jnp.float32))
                sc_v[pl.ds(g * LANES, LANES)] = jnp.exp(
                    acc * half_expw + c_row)
                return c2

            lax.fori_loop(0, NB // LANES, group_body, 0)
            pltpu.sync_copy(sc_v, out_hbm.at[pl.ds(blk * NB, NB)])
            return carry

        lax.fori_loop(0, n_blocks, block_body, 0)

    return sc_kernel


def kernel(pos_x, neg_x, emb_table, pair_w0, c):
    b = pos_x.shape[0]
    pos128 = jnp.pad(pos_x, ((0, 0), (0, 128 - NUM_DOMAINS)), mode="edge")
    neg128 = jnp.pad(neg_x, ((0, 0), (0, 0), (0, 128 - NUM_DOMAINS)),
                     mode="edge")
    x128 = jnp.concatenate(
        [pos128, neg128.reshape(b * NUM_NEG, 128)], axis=0)
    n_rows = x128.shape[0]
    idx1d = x128.reshape(-1)
    wc = jnp.stack([
        jnp.broadcast_to(pair_w0[0], (LANES,)),
        jnp.broadcast_to(c[0], (LANES,)),
    ]).astype(jnp.float32)
    scores = _make_sc_kernel(n_rows)(idx1d, emb_table, wc)
    pos_score = scores[:b]
    neg_score = scores[b:].reshape(b, NUM_NEG)
    return pos_score, neg_score


# speedup vs baseline: 1.1563x; 1.1190x over previous
import functools

import jax
import jax.numpy as jnp
from jax import lax
from jax.experimental import pallas as pl
from jax.experimental.pallas import tpu as pltpu
from jax.experimental.pallas import tpu_sc as plsc

EMB_DIM = 32
NUM_DOMAINS = 10
NUM_NEG = 5
NUM_TILES = 32
NB = 128
LANES = 16


def _make_sc_kernel(n_rows):
    rows_per_tile = n_rows // NUM_TILES
    n_blocks = rows_per_tile // NB
    slots = NB * NUM_DOMAINS

    @functools.partial(
        pl.kernel,
        out_type=jax.ShapeDtypeStruct((n_rows,), jnp.float32),
        mesh=plsc.VectorSubcoreMesh(core_axis_name="c", subcore_axis_name="s"),
        compiler_params=pltpu.CompilerParams(use_tc_tiling_on_sc=False),
        scratch_types=[
            pltpu.VMEM((NB * 128,), jnp.int32),
            pltpu.VMEM((NB, NUM_DOMAINS, EMB_DIM), jnp.float32),
            pltpu.VMEM((2, LANES), jnp.float32),
            pltpu.VMEM((NB,), jnp.float32),
            pltpu.SemaphoreType.DMA,
        ],
    )
    def sc_kernel(idx_hbm, table_hbm, wc_hbm, out_hbm, idx_v, rows_v, wc_v,
                  sc_v, sem):
        wid = lax.axis_index("s") * 2 + lax.axis_index("c")
        pltpu.sync_copy(wc_hbm, wc_v)
        w_row = wc_v[0, :]
        c_row = wc_v[1, :]
        half_expw = jnp.exp(w_row) * 0.5

        def block_body(b, carry):
            blk = wid * n_blocks + b
            pltpu.sync_copy(idx_hbm.at[pl.ds(blk * NB * 128, NB * 128)], idx_v)
            copies = [
                pltpu.async_copy(
                    table_hbm.at[idx_v.at[pl.ds(r * 128, NUM_DOMAINS)]],
                    rows_v.at[r], sem)
                for r in range(NB)
            ]
            for cp in copies:
                cp.wait()

            lane = lax.iota(jnp.int32, LANES)

            def group_body(g, c2):
                def row_body(ii, acc):
                    row = g * LANES + ii
                    v0 = rows_v[row, 0, 0:LANES]
                    v1 = rows_v[row, 0, LANES:EMB_DIM]
                    s0 = v0
                    s1 = v1
                    q = v0 * v0 + v1 * v1
                    for j in range(1, NUM_DOMAINS):
                        v0 = rows_v[row, j, 0:LANES]
                        v1 = rows_v[row, j, LANES:EMB_DIM]
                        s0 = s0 + v0
                        s1 = s1 + v1
                        q = q + v0 * v0 + v1 * v1
                    t = s0 * s0 + s1 * s1 - q
                    for step in (8, 4, 2, 1):
                        t = t + t.at[lane ^ step].get(
                            mode="promise_in_bounds")
                    return jnp.where(lane == ii, t, acc)

                acc = lax.fori_loop(0, LANES, row_body,
                                    jnp.zeros((LANES,), jnp.float32))
                sc_v[pl.ds(g * LANES, LANES)] = jnp.exp(
                    acc * half_expw + c_row)
                return c2

            lax.fori_loop(0, NB // LANES, group_body, 0)
            pltpu.sync_copy(sc_v, out_hbm.at[pl.ds(blk * NB, NB)])
            return carry

        lax.fori_loop(0, n_blocks, block_body, 0)

    return sc_kernel


def kernel(pos_x, neg_x, emb_table, pair_w0, c):
    b = pos_x.shape[0]
    pos128 = jnp.pad(pos_x, ((0, 0), (0, 128 - NUM_DOMAINS)))
    neg128 = jnp.pad(neg_x, ((0, 0), (0, 0), (0, 128 - NUM_DOMAINS)))
    x128 = jnp.concatenate(
        [pos128, neg128.reshape(b * NUM_NEG, 128)], axis=0)
    n_rows = x128.shape[0]
    idx1d = x128.reshape(-1)
    wc = jnp.stack([
        jnp.broadcast_to(pair_w0[0], (LANES,)),
        jnp.broadcast_to(c[0], (LANES,)),
    ]).astype(jnp.float32)
    scores = _make_sc_kernel(n_rows)(idx1d, emb_table, wc)
    pos_score = scores[:b]
    neg_score = scores[b:].reshape(b, NUM_NEG)
    return pos_score, neg_score
